# Initial kernel scaffold; baseline (speedup 1.0000x reference)
import jax
import jax.numpy as jnp
from jax.experimental import pallas as pl


def kernel(x, Wg, bg, W1, b1, W2, b2):
    raise NotImplementedError("write your pallas kernel here")


# speedup vs baseline: 1.5503x; 1.5503x over previous
import functools
import math

import jax
import jax.numpy as jnp
from jax import lax
from jax.experimental import pallas as pl
from jax.experimental.pallas import tpu as pltpu
from jax.experimental.pallas import tpu_sc as plsc

K = 2
CAP_F = 1.25

NC = 2
NS = 16
NW = NC * NS


def _route_body(cap, dump, x_ref, wg_ref, bg_ref,
                dst0_ref, dst1_ref, src0_ref, src1_ref, w0_ref, w1_ref):
    T, H = x_ref.shape
    E = wg_ref.shape[1]
    logits = jnp.dot(x_ref[...], wg_ref[...],
                     preferred_element_type=jnp.float32) + bg_ref[...]
    eidx = lax.broadcasted_iota(jnp.int32, (T, E), 1)

    m0 = jnp.max(logits, axis=1, keepdims=True)
    am0 = jnp.min(jnp.where(logits == m0, eidx, E), axis=1, keepdims=True)
    l2 = jnp.where(eidx == am0, -jnp.inf, logits)
    m1 = jnp.max(l2, axis=1, keepdims=True)
    am1 = jnp.min(jnp.where(l2 == m1, eidx, E), axis=1, keepdims=True)

    e1 = jnp.exp(m1 - m0)
    w0 = 1.0 / (1.0 + e1)
    w1 = e1 / (1.0 + e1)

    oh0 = (eidx == am0).astype(jnp.float32)
    oh1 = (eidx == am1).astype(jnp.float32)
    rowcnt = oh0 + oh1

    BT = 256
    NB = T // BT
    r = lax.broadcasted_iota(jnp.int32, (BT, BT), 0)
    c = lax.broadcasted_iota(jnp.int32, (BT, BT), 1)
    lstrict = (c < r).astype(jnp.float32)
    carry = jnp.zeros((1, E), jnp.float32)
    capf = jnp.float32(cap)
    for b in range(NB):
        lo, hi = b * BT, (b + 1) * BT
        blk = lax.slice(rowcnt, (lo, 0), (hi, E))
        pref = jnp.dot(lstrict, blk, preferred_element_type=jnp.float32) + carry
        carry = carry + jnp.sum(blk, axis=0, keepdims=True)
        oh0b = lax.slice(oh0, (lo, 0), (hi, E))
        oh1b = lax.slice(oh1, (lo, 0), (hi, E))
        pos0 = jnp.sum(pref * oh0b, axis=1, keepdims=True)
        pos1 = jnp.sum(pref * oh1b, axis=1, keepdims=True)
        am0b = lax.slice(am0, (lo, 0), (hi, 1))
        am1b = lax.slice(am1, (lo, 0), (hi, 1))
        v0 = pos0 < capf
        v1 = pos1 < capf
        slot0 = am0b * cap + pos0.astype(jnp.int32)
        slot1 = am1b * cap + pos1.astype(jnp.int32)
        dst0_ref[lo:hi, :] = jnp.where(v0, slot0, dump)
        dst1_ref[lo:hi, :] = jnp.where(v1, slot1, dump)
        src0_ref[lo:hi, :] = jnp.where(v0, slot0, 0)
        src1_ref[lo:hi, :] = jnp.where(v1, slot1, 0)
        w0b = lax.slice(w0, (lo, 0), (hi, 1))
        w1b = lax.slice(w1, (lo, 0), (hi, 1))
        w0_ref[lo:hi, :] = jnp.where(v0, w0b, 0.0)
        w1_ref[lo:hi, :] = jnp.where(v1, w1b, 0.0)


def _route(x, Wg, bg, cap, dump, interpret=False):
    T = x.shape[0]
    i32 = jax.ShapeDtypeStruct((T, 1), jnp.int32)
    f32 = jax.ShapeDtypeStruct((T, 1), jnp.float32)
    return pl.pallas_call(
        functools.partial(_route_body, cap, dump),
        out_shape=(i32, i32, i32, i32, f32, f32),
        interpret=interpret,
    )(x, Wg, bg.reshape(1, -1))


def _make_dispatch(T, H, rows, tpw, ch):
    nch = tpw // ch
    mesh = plsc.VectorSubcoreMesh(core_axis_name="c", subcore_axis_name="s")

    @functools.partial(
        pl.kernel,
        out_type=jax.ShapeDtypeStruct((rows, H), jnp.float32),
        mesh=mesh,
        scratch_types=[
            pltpu.VMEM((nch, ch), jnp.int32),
            pltpu.VMEM((nch, ch), jnp.int32),
            pltpu.VMEM((ch, H), jnp.float32),
            pltpu.SemaphoreType.DMA,
        ],
    )
    def dispatch(x_hbm, d0_hbm, d1_hbm, ei_hbm, d0_v, d1_v, buf, sem):
        wid = lax.axis_index("s") * NC + lax.axis_index("c")
        pltpu.sync_copy(d0_hbm.at[wid], d0_v)
        pltpu.sync_copy(d1_hbm.at[wid], d1_v)
        for j in range(nch):
            base = wid * tpw + j * ch
            pltpu.sync_copy(x_hbm.at[pl.ds(base, ch)], buf)
            pltpu.async_copy(buf, ei_hbm.at[d0_v.at[j]], sem).wait()
            pltpu.async_copy(buf, ei_hbm.at[d1_v.at[j]], sem).wait()

    return dispatch


def _ffn_body(ei_ref, w1_ref, b1_ref, w2_ref, b2_ref, out_ref):
    f = pl.program_id(1)
    h = jnp.dot(ei_ref[...], w1_ref[0], preferred_element_type=jnp.float32)
    h = jnp.maximum(h + b1_ref[0], 0.0)
    part = jnp.dot(h, w2_ref[0], preferred_element_type=jnp.float32)

    @pl.when(f == 0)
    def _():
        out_ref[...] = part + b2_ref[0]

    @pl.when(f > 0)
    def _():
        out_ref[...] += part


def _ffn(ei, W1, b1, W2, b2, cap, fb=512, interpret=False):
    E, H, F = W1.shape
    nf = F // fb
    return pl.pallas_call(
        _ffn_body,
        grid=(E, nf),
        in_specs=[
            pl.BlockSpec((cap, H), lambda e, f: (e, 0)),
            pl.BlockSpec((1, H, fb), lambda e, f: (e, 0, f)),
            pl.BlockSpec((1, 1, fb), lambda e, f: (e, 0, f)),
            pl.BlockSpec((1, fb, H), lambda e, f: (e, f, 0)),
            pl.BlockSpec((1, 1, H), lambda e, f: (e, 0, 0)),
        ],
        out_specs=pl.BlockSpec((cap, H), lambda e, f: (e, 0)),
        out_shape=jax.ShapeDtypeStruct((E * cap, H), jnp.float32),
        interpret=interpret,
    )(ei, W1, b1.reshape(E, 1, F), W2, b2.reshape(E, 1, H))


def _make_gather2(T, H, rows, tpw, ch):
    nch = tpw // ch
    mesh = plsc.VectorSubcoreMesh(core_axis_name="c", subcore_axis_name="s")
    out = jax.ShapeDtypeStruct((T, H), jnp.float32)

    @functools.partial(
        pl.kernel,
        out_type=(out, out),
        mesh=mesh,
        scratch_types=[
            pltpu.VMEM((nch, ch), jnp.int32),
            pltpu.VMEM((nch, ch), jnp.int32),
            pltpu.VMEM((ch, H), jnp.float32),
            pltpu.SemaphoreType.DMA,
        ],
    )
    def gather2(eo_hbm, s0_hbm, s1_hbm, g0_hbm, g1_hbm, s0_v, s1_v, buf, sem):
        wid = lax.axis_index("s") * NC + lax.axis_index("c")
        pltpu.sync_copy(s0_hbm.at[wid], s0_v)
        pltpu.sync_copy(s1_hbm.at[wid], s1_v)
        for j in range(nch):
            base = wid * tpw + j * ch
            pltpu.async_copy(eo_hbm.at[s0_v.at[j]], buf, sem).wait()
            pltpu.sync_copy(buf, g0_hbm.at[pl.ds(base, ch)])
            pltpu.async_copy(eo_hbm.at[s1_v.at[j]], buf, sem).wait()
            pltpu.sync_copy(buf, g1_hbm.at[pl.ds(base, ch)])

    return gather2


def _combine_body(g0_ref, g1_ref, w0_ref, w1_ref, out_ref):
    out_ref[...] = w0_ref[...] * g0_ref[...] + w1_ref[...] * g1_ref[...]


def _combine(g0, g1, w0, w1, rb=512, interpret=False):
    T, H = g0.shape
    return pl.pallas_call(
        _combine_body,
        grid=(T // rb,),
        in_specs=[
            pl.BlockSpec((rb, H), lambda i: (i, 0)),
            pl.BlockSpec((rb, H), lambda i: (i, 0)),
            pl.BlockSpec((rb, 1), lambda i: (i, 0)),
            pl.BlockSpec((rb, 1), lambda i: (i, 0)),
        ],
        out_specs=pl.BlockSpec((rb, H), lambda i: (i, 0)),
        out_shape=jax.ShapeDtypeStruct((T, H), jnp.float32),
        interpret=interpret,
    )(g0, g1, w0, w1)


def kernel(x, Wg, bg, W1, b1, W2, b2):
    T, H = x.shape
    E = Wg.shape[1]
    cap = int(math.ceil(T * K / E * CAP_F))
    dump = E * cap
    rows = E * cap + 8

    dst0, dst1, src0, src1, w0, w1 = _route(x, Wg, bg, cap, dump)

    tpw = T // NW
    ch = 32
    d0 = dst0.reshape(NW, tpw // ch, ch)
    d1 = dst1.reshape(NW, tpw // ch, ch)
    ei = _make_dispatch(T, H, rows, tpw, ch)(x, d0, d1)

    eo = _ffn(ei, W1, b1, W2, b2, cap)

    s0 = src0.reshape(NW, tpw // ch, ch)
    s1 = src1.reshape(NW, tpw // ch, ch)
    g0, g1 = _make_gather2(T, H, E * cap, tpw, ch)(eo, s0, s1)

    return _combine(g0, g1, w0.reshape(T, 1), w1.reshape(T, 1))
